# Initial kernel scaffold; baseline (speedup 1.0000x reference)
import jax
import jax.numpy as jnp
from jax.experimental import pallas as pl


def kernel(tokens, embedding_weight):
    raise NotImplementedError("write your pallas kernel here")


# speedup vs baseline: 1.0441x; 1.0441x over previous
import functools
import math

import jax
import jax.numpy as jnp
from jax import lax
from jax.experimental import pallas as pl
from jax.experimental.pallas import tpu as pltpu
from jax.experimental.pallas import tpu_sc as plsc

_D = 32
_SCALE = math.sqrt(32.0)
_NC, _NS = 2, 16
_NW = _NC * _NS
_CHUNK = 128


@functools.lru_cache(maxsize=None)
def _build(n_tok: int):
    per_w = n_tok // _NW
    nchunk = per_w // _CHUNK
    assert per_w * _NW == n_tok and nchunk * _CHUNK == per_w

    mesh = plsc.VectorSubcoreMesh(core_axis_name="c", subcore_axis_name="s")

    @functools.partial(
        pl.kernel,
        out_type=jax.ShapeDtypeStruct((_NW, per_w, _D), jnp.float32),
        mesh=mesh,
        scratch_types=[
            pltpu.VMEM((nchunk, _CHUNK), jnp.int32),
            pltpu.VMEM((_CHUNK, _D), jnp.float32),
            pltpu.SemaphoreType.DMA,
        ],
        compiler_params=pltpu.CompilerParams(use_tc_tiling_on_sc=False),
    )
    def emb(tok_hbm, table_hbm, out_hbm, idx_v, rows_v, gsem):
        w = lax.axis_index("s") * _NC + lax.axis_index("c")
        pltpu.sync_copy(tok_hbm.at[w], idx_v)

        @pl.loop(0, nchunk)
        def _step(j):
            pltpu.async_copy(table_hbm.at[idx_v.at[j]], rows_v, gsem).wait()

            @pl.loop(0, _CHUNK)
            def _srow(r):
                for c in range(_D // 16):
                    rows_v[r, pl.ds(c * 16, 16)] = (
                        rows_v[r, pl.ds(c * 16, 16)] * _SCALE
                    )

            pltpu.sync_copy(rows_v, out_hbm.at[w, pl.ds(j * _CHUNK, _CHUNK)])

    return emb


def kernel(tokens, embedding_weight):
    n_rows, n_cols = tokens.shape
    n_tok = n_rows * n_cols
    flat = tokens.astype(jnp.int32).reshape(_NW, n_tok // _NW // _CHUNK, _CHUNK)
    out = _build(n_tok)(flat, embedding_weight)
    return out.reshape(n_rows, n_cols, _D)


# speedup vs baseline: 1.2360x; 1.1837x over previous
import functools
import math

import jax
import jax.numpy as jnp
from jax import lax
from jax.experimental import pallas as pl
from jax.experimental.pallas import tpu as pltpu
from jax.experimental.pallas import tpu_sc as plsc

_D = 32
_SCALE = math.sqrt(32.0)
_NC, _NS = 2, 16
_NW = _NC * _NS
_CHUNK = 128
_G = 10


@functools.lru_cache(maxsize=None)
def _build(n_tok: int):
    per_w = n_tok // _NW
    nchunk = per_w // _CHUNK
    ngroup = nchunk // _G
    grows = _G * _CHUNK
    assert per_w * _NW == n_tok and ngroup * _G == nchunk and ngroup % 2 == 0

    mesh = plsc.VectorSubcoreMesh(core_axis_name="c", subcore_axis_name="s")

    @functools.partial(
        pl.kernel,
        out_type=jax.ShapeDtypeStruct((_NW, per_w, _D), jnp.float32),
        mesh=mesh,
        scratch_types=[
            pltpu.VMEM((nchunk, _CHUNK), jnp.int32),
            pltpu.VMEM((2, grows, _D), jnp.float32),
            pltpu.SemaphoreType.DMA,
            pltpu.SemaphoreType.DMA,
            pltpu.SemaphoreType.DMA,
            pltpu.SemaphoreType.DMA,
        ],
        compiler_params=pltpu.CompilerParams(use_tc_tiling_on_sc=False),
    )
    def emb(tok_hbm, table_hbm, out_hbm, idx_v, rows_v, g0, g1, o0, o1):
        w = lax.axis_index("s") * _NC + lax.axis_index("c")
        gsems = (g0, g1)
        osems = (o0, o1)
        pltpu.sync_copy(tok_hbm.at[w], idx_v)

        def issue_gathers(t, p):
            for b in range(_G):
                pltpu.async_copy(
                    table_hbm.at[idx_v.at[t * _G + b]],
                    rows_v.at[p, pl.ds(b * _CHUNK, _CHUNK)],
                    gsems[p],
                )

        def drain(sem, p):
            pltpu.make_async_copy(
                out_hbm.at[w, pl.ds(0, grows)], rows_v.at[p], sem
            ).wait()

        issue_gathers(0, 0)

        @pl.loop(0, ngroup, step=2)
        def _group2(t0):
            for dp in range(2):
                t = t0 + dp
                p = dp
                drain(gsems[p], p)

                @pl.when(t + 1 < ngroup)
                def _():
                    @pl.when(t >= 1)
                    def _():
                        drain(osems[1 - p], 1 - p)

                    issue_gathers(t + 1, 1 - p)

                @pl.loop(0, grows, unroll=8)
                def _srow(r):
                    for c in range(_D // 16):
                        rows_v[p, r, pl.ds(c * 16, 16)] = (
                            rows_v[p, r, pl.ds(c * 16, 16)] * _SCALE
                        )

                pltpu.async_copy(
                    rows_v.at[p],
                    out_hbm.at[w, pl.ds(t * grows, grows)],
                    osems[p],
                )

        drain(osems[0], 0)
        drain(osems[1], 1)

    return emb


def kernel(tokens, embedding_weight):
    n_rows, n_cols = tokens.shape
    n_tok = n_rows * n_cols
    flat = tokens.astype(jnp.int32).reshape(_NW, n_tok // _NW // _CHUNK, _CHUNK)
    out = _build(n_tok)(flat, embedding_weight)
    return out.reshape(n_rows, n_cols, _D)


# speedup vs baseline: 1.6996x; 1.3751x over previous
import functools
import math

import jax
import jax.numpy as jnp
from jax import lax
from jax.experimental import pallas as pl
from jax.experimental.pallas import tpu as pltpu
from jax.experimental.pallas import tpu_sc as plsc

_D = 32
_SCALE = math.sqrt(32.0)
_NC, _NS = 2, 16
_NW = _NC * _NS
_G = 16


@functools.lru_cache(maxsize=None)
def _build(n_rows: int, n_cols: int):
    rows_per_w = n_rows // _NW
    ngroup = rows_per_w // _G
    assert rows_per_w * _NW == n_rows
    assert ngroup * _G == rows_per_w and ngroup % 2 == 0

    mesh = plsc.VectorSubcoreMesh(core_axis_name="c", subcore_axis_name="s")

    @functools.partial(
        pl.kernel,
        out_type=jax.ShapeDtypeStruct((_NW, rows_per_w, n_cols, _D), jnp.float32),
        mesh=mesh,
        scratch_types=[
            pltpu.VMEM((rows_per_w, n_cols), jnp.int32),
            pltpu.VMEM((2, _G, n_cols, _D), jnp.float32),
            pltpu.SemaphoreType.DMA,
            pltpu.SemaphoreType.DMA,
            pltpu.SemaphoreType.DMA,
            pltpu.SemaphoreType.DMA,
        ],
        compiler_params=pltpu.CompilerParams(use_tc_tiling_on_sc=False),
    )
    def emb(tok_hbm, table_hbm, out_hbm, idx_v, rows_v, g0, g1, o0, o1):
        w = lax.axis_index("s") * _NC + lax.axis_index("c")
        gsems = (g0, g1)
        osems = (o0, o1)
        pltpu.sync_copy(tok_hbm.at[w], idx_v)

        def issue_gathers(t, p):
            for b in range(_G):
                pltpu.async_copy(
                    table_hbm.at[idx_v.at[t * _G + b]],
                    rows_v.at[p, b],
                    gsems[p],
                )

        def drain(sem, p):
            pltpu.make_async_copy(
                out_hbm.at[w, pl.ds(0, _G)], rows_v.at[p], sem
            ).wait()

        issue_gathers(0, 0)

        @pl.loop(0, ngroup, step=2)
        def _group2(t0):
            for dp in range(2):
                t = t0 + dp
                p = dp
                drain(gsems[p], p)

                @pl.when(t + 1 < ngroup)
                def _():
                    @pl.when(t >= 1)
                    def _():
                        drain(osems[1 - p], 1 - p)

                    issue_gathers(t + 1, 1 - p)

                @pl.loop(0, _G)
                def _srow(a):
                    @pl.loop(0, n_cols, unroll=5)
                    def _scol(b):
                        for c in range(_D // 16):
                            rows_v[p, a, b, pl.ds(c * 16, 16)] = (
                                rows_v[p, a, b, pl.ds(c * 16, 16)] * _SCALE
                            )

                pltpu.async_copy(
                    rows_v.at[p],
                    out_hbm.at[w, pl.ds(t * _G, _G)],
                    osems[p],
                )

        drain(osems[0], 0)
        drain(osems[1], 1)

    return emb


def kernel(tokens, embedding_weight):
    n_rows, n_cols = tokens.shape
    toks = tokens.astype(jnp.int32).reshape(_NW, n_rows // _NW, n_cols)
    out = _build(n_rows, n_cols)(toks, embedding_weight)
    return out.reshape(n_rows, n_cols, _D)
